# baseline (speedup 1.0000x reference)
import jax
import jax.numpy as jnp
from jax import lax
from jax.experimental import pallas as pl
from jax.experimental.pallas import tpu as pltpu
from jax.experimental.pallas import tpu_sc as plsc

L = 16
C = 16
NW = 32
B = 4
HO = 512
WO = 512
HOWO = HO * WO
N = B * HOWO
NPW = N // NW
K = 256
NCHUNK = NPW // K
SIZES = (1024, 512, 256, 128)
ISLICE = 128
HIMASK = -65536


def _coords(xv, yv, size):
    gx = xv * 2.0 - 1.0
    gy = yv * 2.0 - 1.0
    fx = (gx + 1.0) * 0.5 * float(size - 1)
    fy = (gy + 1.0) * 0.5 * float(size - 1)
    xi = fx.astype(jnp.int32)
    yi = fy.astype(jnp.int32)
    return fx, fy, xi, yi


def _body(xyf, mskf, t1, t2, t3, t4, out,
          xy_v, msk_v, i1, i2, i3, i4, r1, r2, r3, r4, acc,
          sg0, sg1, si0, si1, so0, so1):
    tabs = (t1, t2, t3, t4)
    idxs = (i1, i2, i3, i4)
    rows = (r1, r2, r3, r4)
    sems_g = (sg0, sg1)
    sems_i = (si0, si1)
    sems_o = (so0, so1)
    cid = lax.axis_index("c")
    sid = lax.axis_index("s")
    wid = sid * 2 + cid
    batch = wid // (NW // B)
    obase = (wid % (NW // B)) * NPW
    iota = lax.iota(jnp.int32, L)

    def fire_in(t, p):
        gp = wid * NPW + t * K
        pltpu.async_copy(xyf.at[pl.ds(2 * gp, 2 * K)], xy_v.at[p], sems_i[p])
        pltpu.async_copy(mskf.at[pl.ds(gp, K)], msk_v.at[p], sems_i[p])

    def drain_in(p):
        pltpu.make_async_copy(xyf.at[pl.ds(0, 2 * K)], xy_v.at[p],
                              sems_i[p]).wait()
        pltpu.make_async_copy(mskf.at[pl.ds(0, K)], msk_v.at[p],
                              sems_i[p]).wait()

    def load_xy(p, point):
        xv = plsc.load_gather(xy_v.at[p], [point * 2])
        yv = plsc.load_gather(xy_v.at[p], [point * 2 + 1])
        return xv, yv

    def pass_a(p):
        def grp_a(g, c2):
            xv, yv = load_xy(p, g * L + iota)
            for lvl, size in enumerate(SIZES):
                _, _, xi, yi = _coords(xv, yv, size)
                xi = jnp.minimum(xi, size - 2)
                yi = jnp.minimum(yi, size - 2)
                r00 = yi * size + xi
                idx_l = idxs[lvl]
                idx_l[p, pl.ds(0 * K + g * L, L)] = r00
                idx_l[p, pl.ds(1 * K + g * L, L)] = r00 + size
            return c2
        lax.fori_loop(0, K // L, grp_a, 0)

    def fire_gathers(p):
        for lvl in range(4):
            for j in range(2 * K // ISLICE):
                pltpu.async_copy(
                    tabs[lvl].at[idxs[lvl].at[p, pl.ds(j * ISLICE, ISLICE)]],
                    rows[lvl].at[p, pl.ds(j * ISLICE, ISLICE)],
                    sems_g[p])

    def drain_gathers(p):
        for lvl in range(4):
            pltpu.make_async_copy(
                tabs[lvl].at[idxs[lvl].at[p]],
                rows[lvl].at[p],
                sems_g[p]).wait()

    def combine(p):
        @plsc.parallel_loop(0, K // L, 1)
        def grp_b(g):
            point = g * L + iota
            xv, yv = load_xy(p, point)
            mv = msk_v[p, pl.ds(g * L, L)]
            ws = []
            for size in SIZES:
                fx, fy, xi, yi = _coords(xv, yv, size)
                wx1 = fx - xi.astype(jnp.float32)
                wy1 = fy - yi.astype(jnp.float32)
                wx0 = 1.0 - wx1
                wy0 = 1.0 - wy1
                ws.append(((wy0 * wx0, wy0 * wx1), (wy1 * wx0, wy1 * wx1)))
            rys = (point, point + K)
            @plsc.parallel_loop(0, C, 1, unroll=4)
            def _dloop(d):
                cold = (d + iota) & (C - 1)
                terms = []
                for lvl in range(4):
                    for yc in range(2):
                        w = plsc.load_gather(rows[lvl].at[p], [rys[yc], cold])
                        v0 = plsc.bitcast(w << 16, jnp.float32)
                        v1 = plsc.bitcast(w & HIMASK, jnp.float32)
                        terms.append(ws[lvl][yc][0] * v0
                                     + ws[lvl][yc][1] * v1)
                while len(terms) > 1:
                    terms = [terms[i] + terms[i + 1]
                             for i in range(0, len(terms), 2)]
                plsc.store_scatter(acc.at[p], [cold, point], terms[0] * mv)

    def fire_out(t, p):
        pltpu.async_copy(acc.at[p], out.at[batch, :, pl.ds(obase + t * K, K)],
                         sems_o[p])

    def drain_out(p):
        pltpu.make_async_copy(acc.at[p], out.at[batch, :, pl.ds(obase, K)],
                              sems_o[p]).wait()

    fire_in(0, 0)
    drain_in(0)
    pass_a(0)
    fire_gathers(0)

    def body(tt, carry):
        for p in (0, 1):
            t = 2 * tt + p
            q = 1 - p
            if p == 0:
                fire_in(t + 1, q)
            else:
                @pl.when(t + 1 < NCHUNK)
                def _():
                    fire_in(t + 1, q)
            drain_gathers(p)

            @pl.when(tt >= 1)
            def _():
                drain_out(p)
            combine(p)
            fire_out(t, p)

            if p == 0:
                drain_in(q)
                pass_a(q)
                fire_gathers(q)
            else:
                @pl.when(t + 1 < NCHUNK)
                def _():
                    drain_in(q)
                    pass_a(q)
                    fire_gathers(q)
        return carry

    lax.fori_loop(0, NCHUNK // 2, body, 0)
    drain_out(0)
    drain_out(1)


def _pack_tab(layer):
    t = layer[0]
    lo = jax.lax.bitcast_convert_type(t.astype(jnp.bfloat16),
                                      jnp.uint16).astype(jnp.uint32)
    w = lo | (jnp.roll(lo, -1, axis=2) << 16)
    w = jax.lax.bitcast_convert_type(w, jnp.int32)
    return jnp.transpose(w, (1, 2, 0)).reshape(-1, C)


def kernel(x, x_msk, layer1, layer2, layer3, layer4):
    xyf = x.reshape(-1)
    mskf = x_msk.reshape(-1)
    tabs = [_pack_tab(l) for l in (layer1, layer2, layer3, layer4)]

    mesh = plsc.VectorSubcoreMesh(core_axis_name="c", subcore_axis_name="s")
    func = pl.kernel(
        _body,
        out_type=jax.ShapeDtypeStruct((B, C, HOWO), jnp.float32),
        mesh=mesh,
        compiler_params=pltpu.CompilerParams(
            needs_layout_passes=False, use_tc_tiling_on_sc=False),
        scratch_types=[
            pltpu.VMEM((2, 2 * K), jnp.float32),
            pltpu.VMEM((2, K), jnp.float32),
            pltpu.VMEM((2, 2 * K), jnp.int32),
            pltpu.VMEM((2, 2 * K), jnp.int32),
            pltpu.VMEM((2, 2 * K), jnp.int32),
            pltpu.VMEM((2, 2 * K), jnp.int32),
            pltpu.VMEM((2, 2 * K, C), jnp.int32),
            pltpu.VMEM((2, 2 * K, C), jnp.int32),
            pltpu.VMEM((2, 2 * K, C), jnp.int32),
            pltpu.VMEM((2, 2 * K, C), jnp.int32),
            pltpu.VMEM((2, C, K), jnp.float32),
            pltpu.SemaphoreType.DMA,
            pltpu.SemaphoreType.DMA,
            pltpu.SemaphoreType.DMA,
            pltpu.SemaphoreType.DMA,
            pltpu.SemaphoreType.DMA,
            pltpu.SemaphoreType.DMA,
        ],
    )
    y = func(xyf, mskf, *tabs)
    return y.reshape(B, C, HO, WO)


# speedup vs baseline: 1.7162x; 1.7162x over previous
import jax
import jax.numpy as jnp
from jax import lax
from jax.experimental import pallas as pl
from jax.experimental.pallas import tpu as pltpu
from jax.experimental.pallas import tpu_sc as plsc

L = 16
C = 16
NW = 32
B = 4
HO = 512
WO = 512
HOWO = HO * WO
N = B * HOWO
NPW = N // NW
K = 256
NCHUNK = NPW // K
SIZES = (1024, 512, 256, 128)
ISLICE = 128
HIMASK = -65536


def _coords(xv, yv, size):
    gx = xv * 2.0 - 1.0
    gy = yv * 2.0 - 1.0
    fx = (gx + 1.0) * 0.5 * float(size - 1)
    fy = (gy + 1.0) * 0.5 * float(size - 1)
    xi = fx.astype(jnp.int32)
    yi = fy.astype(jnp.int32)
    return fx, fy, xi, yi


def _body(inp, t1, t2, t3, t4, out,
          in_v, i1, i2, i3, i4, r1, r2, r3, r4, acc,
          sg0, sg1, si0, si1, so0, so1):
    tabs = (t1, t2, t3, t4)
    idxs = (i1, i2, i3, i4)
    rows = (r1, r2, r3, r4)
    sems_g = (sg0, sg1)
    sems_i = (si0, si1)
    sems_o = (so0, so1)
    cid = lax.axis_index("c")
    sid = lax.axis_index("s")
    wid = sid * 2 + cid
    batch = wid // (NW // B)
    obase = (wid % (NW // B)) * NPW
    iota = lax.iota(jnp.int32, L)

    def fire_in(t, p):
        gp = wid * NPW + t * K
        pltpu.async_copy(inp.at[:, pl.ds(gp, K)], in_v.at[p], sems_i[p])

    def drain_in(p):
        pltpu.make_async_copy(inp.at[:, pl.ds(0, K)], in_v.at[p],
                              sems_i[p]).wait()

    def pass_a(p):
        def grp_a(g, c2):
            xv = in_v[p, 0, pl.ds(g * L, L)]
            yv = in_v[p, 1, pl.ds(g * L, L)]
            for lvl, size in enumerate(SIZES):
                _, _, xi, yi = _coords(xv, yv, size)
                xi = jnp.minimum(xi, size - 2)
                yi = jnp.minimum(yi, size - 2)
                r00 = yi * size + xi
                idx_l = idxs[lvl]
                idx_l[p, pl.ds(0 * K + g * L, L)] = r00
                idx_l[p, pl.ds(1 * K + g * L, L)] = r00 + size
            return c2
        lax.fori_loop(0, K // L, grp_a, 0)

    def fire_gathers(p):
        for lvl in range(4):
            for j in range(2 * K // ISLICE):
                pltpu.async_copy(
                    tabs[lvl].at[idxs[lvl].at[p, pl.ds(j * ISLICE, ISLICE)]],
                    rows[lvl].at[p, pl.ds(j * ISLICE, ISLICE)],
                    sems_g[p])

    def drain_gathers(p):
        for lvl in range(4):
            pltpu.make_async_copy(
                tabs[lvl].at[idxs[lvl].at[p]],
                rows[lvl].at[p],
                sems_g[p]).wait()

    def combine(p):
        @plsc.parallel_loop(0, K // L, 1)
        def grp_b(g):
            xv = in_v[p, 0, pl.ds(g * L, L)]
            yv = in_v[p, 1, pl.ds(g * L, L)]
            mv = in_v[p, 2, pl.ds(g * L, L)]
            ws = []
            for size in SIZES:
                fx, fy, xi, yi = _coords(xv, yv, size)
                wx1 = fx - xi.astype(jnp.float32)
                wy1 = fy - yi.astype(jnp.float32)
                wx0 = 1.0 - wx1
                wy0 = 1.0 - wy1
                ws.append(((wy0 * wx0, wy0 * wx1), (wy1 * wx0, wy1 * wx1)))
            point = g * L + iota
            rys = (point, point + K)
            @plsc.parallel_loop(0, C, 1, unroll=4)
            def _dloop(d):
                cold = (d + iota) & (C - 1)
                terms = []
                for lvl in range(4):
                    for yc in range(2):
                        w = plsc.load_gather(rows[lvl].at[p], [rys[yc], cold])
                        v0 = plsc.bitcast(w << 16, jnp.float32)
                        v1 = plsc.bitcast(w & HIMASK, jnp.float32)
                        terms.append(ws[lvl][yc][0] * v0
                                     + ws[lvl][yc][1] * v1)
                while len(terms) > 1:
                    terms = [terms[i] + terms[i + 1]
                             for i in range(0, len(terms), 2)]
                plsc.store_scatter(acc.at[p], [cold, point], terms[0] * mv)

    def fire_out(t, p):
        pltpu.async_copy(acc.at[p], out.at[batch, :, pl.ds(obase + t * K, K)],
                         sems_o[p])

    def drain_out(p):
        pltpu.make_async_copy(acc.at[p], out.at[batch, :, pl.ds(obase, K)],
                              sems_o[p]).wait()

    fire_in(0, 0)
    drain_in(0)
    pass_a(0)
    fire_gathers(0)

    def body(tt, carry):
        for p in (0, 1):
            t = 2 * tt + p
            q = 1 - p
            if p == 0:
                fire_in(t + 1, q)
            else:
                @pl.when(t + 1 < NCHUNK)
                def _():
                    fire_in(t + 1, q)
            drain_gathers(p)

            @pl.when(tt >= 1)
            def _():
                drain_out(p)
            combine(p)
            fire_out(t, p)

            if p == 0:
                drain_in(q)
                pass_a(q)
                fire_gathers(q)
            else:
                @pl.when(t + 1 < NCHUNK)
                def _():
                    drain_in(q)
                    pass_a(q)
                    fire_gathers(q)
        return carry

    lax.fori_loop(0, NCHUNK // 2, body, 0)
    drain_out(0)
    drain_out(1)


def _pack_tab(layer):
    t = layer[0]
    lo = jax.lax.bitcast_convert_type(t.astype(jnp.bfloat16),
                                      jnp.uint16).astype(jnp.uint32)
    w = lo | (jnp.roll(lo, -1, axis=2) << 16)
    w = jax.lax.bitcast_convert_type(w, jnp.int32)
    return jnp.transpose(w, (1, 2, 0)).reshape(-1, C)


def kernel(x, x_msk, layer1, layer2, layer3, layer4):
    inp = jnp.stack([x[..., 0].reshape(-1), x[..., 1].reshape(-1),
                     x_msk.reshape(-1)])
    tabs = [_pack_tab(l) for l in (layer1, layer2, layer3, layer4)]

    mesh = plsc.VectorSubcoreMesh(core_axis_name="c", subcore_axis_name="s")
    func = pl.kernel(
        _body,
        out_type=jax.ShapeDtypeStruct((B, C, HOWO), jnp.float32),
        mesh=mesh,
        compiler_params=pltpu.CompilerParams(
            needs_layout_passes=False, use_tc_tiling_on_sc=False),
        scratch_types=[
            pltpu.VMEM((2, 3, K), jnp.float32),
            pltpu.VMEM((2, 2 * K), jnp.int32),
            pltpu.VMEM((2, 2 * K), jnp.int32),
            pltpu.VMEM((2, 2 * K), jnp.int32),
            pltpu.VMEM((2, 2 * K), jnp.int32),
            pltpu.VMEM((2, 2 * K, C), jnp.int32),
            pltpu.VMEM((2, 2 * K, C), jnp.int32),
            pltpu.VMEM((2, 2 * K, C), jnp.int32),
            pltpu.VMEM((2, 2 * K, C), jnp.int32),
            pltpu.VMEM((2, C, K), jnp.float32),
            pltpu.SemaphoreType.DMA,
            pltpu.SemaphoreType.DMA,
            pltpu.SemaphoreType.DMA,
            pltpu.SemaphoreType.DMA,
            pltpu.SemaphoreType.DMA,
            pltpu.SemaphoreType.DMA,
        ],
    )
    y = func(inp, *tabs)
    return y.reshape(B, C, HO, WO)
